# Initial kernel scaffold; baseline (speedup 1.0000x reference)
import functools

import jax
import jax.numpy as jnp
from jax import lax
from jax.experimental import pallas as pl
from jax.experimental.pallas import tpu as pltpu
from jax.experimental.pallas import tpu_sc as plsc

NC = 2
NS = 16
NW = NC * NS

CHUNK = 1024


def _make_kernel(B, V, D):
    assert B % NW == 0
    b_per_w = B // NW
    assert b_per_w % CHUNK == 0
    n_chunks = b_per_w // CHUNK

    mesh = plsc.VectorSubcoreMesh(core_axis_name="c", subcore_axis_name="s")

    @functools.partial(
        pl.kernel,
        out_type=jax.ShapeDtypeStruct((B, D), jnp.float32),
        mesh=mesh,
        scratch_types=[
            pltpu.VMEM((CHUNK,), jnp.int32),
            pltpu.VMEM((CHUNK, D), jnp.float32),
            pltpu.SemaphoreType.DMA,
            pltpu.SemaphoreType.DMA,
            pltpu.SemaphoreType.DMA,
        ],
    )
    def gather_kernel(x_hbm, w_hbm, out_hbm, idx_v, rows_v, sem_i, sem_g, sem_o):
        wid = lax.axis_index("s") * NC + lax.axis_index("c")
        base = wid * b_per_w

        def step(g, _):
            off = base + g * CHUNK
            pltpu.async_copy(x_hbm.at[pl.ds(off, CHUNK)], idx_v, sem_i).wait()
            pltpu.async_copy(w_hbm.at[idx_v], rows_v, sem_g).wait()
            pltpu.async_copy(rows_v, out_hbm.at[pl.ds(off, CHUNK)], sem_o).wait()
            return 0

        lax.fori_loop(0, n_chunks, step, 0)

    return gather_kernel


def kernel(x, W):
    B_, H = x.shape
    V, D = W.shape
    flat = x.reshape(-1).astype(jnp.int32)
    out = _make_kernel(flat.shape[0], V, D)(flat, W)
    return out.reshape(B_, H, D)


# speedup vs baseline: 4.8084x; 4.8084x over previous
import functools

import jax
import jax.numpy as jnp
from jax import lax
from jax.experimental import pallas as pl
from jax.experimental.pallas import tpu as pltpu
from jax.experimental.pallas import tpu_sc as plsc

NC = 2
NS = 16
NW = NC * NS

CHUNK = 1024


def _make_kernel(B, V, D):
    assert B % NW == 0
    b_per_w = B // NW
    assert b_per_w % CHUNK == 0
    n_chunks = b_per_w // CHUNK

    mesh = plsc.VectorSubcoreMesh(core_axis_name="c", subcore_axis_name="s")

    @functools.partial(
        pl.kernel,
        out_type=jax.ShapeDtypeStruct((B, D), jnp.float32),
        mesh=mesh,
        scratch_types=[
            pltpu.VMEM((CHUNK,), jnp.int32),
            pltpu.VMEM((CHUNK, D), jnp.float32),
            pltpu.SemaphoreType.DMA,
            pltpu.SemaphoreType.DMA,
            pltpu.SemaphoreType.DMA,
        ],
        compiler_params=pltpu.CompilerParams(use_tc_tiling_on_sc=False),
    )
    def gather_kernel(x_hbm, w_hbm, out_hbm, idx_v, rows_v, sem_i, sem_g, sem_o):
        wid = lax.axis_index("s") * NC + lax.axis_index("c")
        base = wid * b_per_w

        def step(g, _):
            off = base + g * CHUNK
            pltpu.async_copy(x_hbm.at[pl.ds(off, CHUNK)], idx_v, sem_i).wait()
            pltpu.async_copy(w_hbm.at[idx_v], rows_v, sem_g).wait()
            pltpu.async_copy(rows_v, out_hbm.at[pl.ds(off, CHUNK)], sem_o).wait()
            return 0

        lax.fori_loop(0, n_chunks, step, 0)

    return gather_kernel


def kernel(x, W):
    B_, H = x.shape
    V, D = W.shape
    flat = x.reshape(-1).astype(jnp.int32)
    out = _make_kernel(flat.shape[0], V, D)(flat, W)
    return out.reshape(B_, H, D)


# speedup vs baseline: 5.0480x; 1.0498x over previous
import functools

import jax
import jax.numpy as jnp
from jax import lax
from jax.experimental import pallas as pl
from jax.experimental.pallas import tpu as pltpu
from jax.experimental.pallas import tpu_sc as plsc

NC = 2
NS = 16
NW = NC * NS

CHUNK = 1024
NBUF = 2


def _make_kernel(B, V, D):
    assert B % NW == 0
    b_per_w = B // NW
    assert b_per_w % CHUNK == 0
    n_chunks = b_per_w // CHUNK
    assert n_chunks % NBUF == 0 and n_chunks >= 2 * NBUF

    mesh = plsc.VectorSubcoreMesh(core_axis_name="c", subcore_axis_name="s")

    @functools.partial(
        pl.kernel,
        out_type=jax.ShapeDtypeStruct((B, D), jnp.float32),
        mesh=mesh,
        scratch_types=[
            pltpu.VMEM((CHUNK,), jnp.int32),
            pltpu.VMEM((CHUNK,), jnp.int32),
            pltpu.VMEM((CHUNK, D), jnp.float32),
            pltpu.VMEM((CHUNK, D), jnp.float32),
            pltpu.SemaphoreType.DMA,
            pltpu.SemaphoreType.DMA,
            pltpu.SemaphoreType.DMA,
            pltpu.SemaphoreType.DMA,
            pltpu.SemaphoreType.DMA,
            pltpu.SemaphoreType.DMA,
        ],
        compiler_params=pltpu.CompilerParams(use_tc_tiling_on_sc=False),
    )
    def gather_kernel(x_hbm, w_hbm, out_hbm, idx0, idx1, rows0, rows1,
                      s_i0, s_i1, s_g0, s_g1, s_o0, s_o1):
        idx_v = [idx0, idx1]
        rows_v = [rows0, rows1]
        sem_i = [s_i0, s_i1]
        sem_g = [s_g0, s_g1]
        sem_o = [s_o0, s_o1]
        wid = lax.axis_index("s") * NC + lax.axis_index("c")
        base = wid * b_per_w

        def idx_copy(g, b):
            return pltpu.make_async_copy(
                x_hbm.at[pl.ds(base + g * CHUNK, CHUNK)], idx_v[b], sem_i[b])

        def gather(b):
            return pltpu.make_async_copy(
                w_hbm.at[idx_v[b]], rows_v[b], sem_g[b])

        def writeout(g, b):
            return pltpu.make_async_copy(
                rows_v[b], out_hbm.at[pl.ds(base + g * CHUNK, CHUNK)], sem_o[b])

        for b in range(NBUF):
            idx_copy(b, b).start()

        def outer(i, _):
            for b in range(NBUF):
                g = i * NBUF + b
                pb = (b - 1) % NBUF

                @pl.when(g >= NBUF)
                def _():
                    writeout(g - NBUF, b).wait()

                idx_copy(g, b).wait()
                gather(b).start()

                @pl.when(g >= 1)
                def _():
                    gather(pb).wait()
                    writeout(g - 1, pb).start()

                @pl.when(jnp.logical_and(g >= 1, g + 1 < n_chunks))
                def _():
                    idx_copy(g + 1, pb).start()
            return 0

        lax.fori_loop(0, n_chunks // NBUF, outer, 0)

        last = n_chunks - 1
        bl = last % NBUF
        gather(bl).wait()
        writeout(last, bl).start()
        for k in range(NBUF - 1, -1, -1):
            writeout(last - k, (last - k) % NBUF).wait()

    return gather_kernel


def kernel(x, W):
    B_, H = x.shape
    V, D = W.shape
    flat = x.reshape(-1).astype(jnp.int32)
    out = _make_kernel(flat.shape[0], V, D)(flat, W)
    return out.reshape(B_, H, D)
